# Initial kernel scaffold; baseline (speedup 1.0000x reference)
import jax
import jax.numpy as jnp
from jax.experimental import pallas as pl


def kernel(des, tweet, num_prop, cat_prop, edge_index, edge_type, W_num, b_num, W_cat, b_cat, W_in, b_in, W_rel, W_root, b_rgcn, W_out1, b_out1, W_out2, b_out2):
    raise NotImplementedError("write your pallas kernel here")


# speedup vs baseline: 4.9652x; 4.9652x over previous
import functools

import jax
import jax.numpy as jnp
from jax import lax
from jax.experimental import pallas as pl
from jax.experimental.pallas import tpu as pltpu
from jax.experimental.pallas import tpu_sc as plsc

NC = 2
NS = 16
EBLK = 2048
CH = 128
NCH = EBLK // CH
ZR = 800
SW = 16


def _leaky(x):
    return jnp.where(x >= 0, x, 0.01 * x)


def _mlp_body(np_ref, cp_ref, wn_ref, bn_ref, wc_ref, bc_ref, wi_ref, bi_ref,
              o_ref):
    nf = _leaky(jnp.dot(np_ref[...], wn_ref[...],
                        preferred_element_type=jnp.float32) + bn_ref[...])
    cf = _leaky(jnp.dot(cp_ref[...], wc_ref[...],
                        preferred_element_type=jnp.float32) + bc_ref[...])
    wi = wi_ref[...]
    h = wi.shape[0] // 2
    x = (jnp.dot(nf, wi[:h], preferred_element_type=jnp.float32)
         + jnp.dot(cf, wi[h:], preferred_element_type=jnp.float32)
         + bi_ref[...])
    o_ref[...] = _leaky(x)


def _input_mlp(num_p, cat_p, wn, bn, wc, bc, wi, bi, n, d, blk):
    grid = (n // blk,)
    fn = num_p.shape[1]
    fc = cat_p.shape[1]
    dh = d // 2
    return pl.pallas_call(
        _mlp_body,
        grid=grid,
        in_specs=[
            pl.BlockSpec((blk, fn), lambda i: (i, 0)),
            pl.BlockSpec((blk, fc), lambda i: (i, 0)),
            pl.BlockSpec((fn, dh), lambda i: (0, 0)),
            pl.BlockSpec((1, dh), lambda i: (0, 0)),
            pl.BlockSpec((fc, dh), lambda i: (0, 0)),
            pl.BlockSpec((1, dh), lambda i: (0, 0)),
            pl.BlockSpec((d, d), lambda i: (0, 0)),
            pl.BlockSpec((1, d), lambda i: (0, 0)),
        ],
        out_specs=pl.BlockSpec((blk, d), lambda i: (i, 0)),
        out_shape=jax.ShapeDtypeStruct((n, d), jnp.float32),
    )(num_p, cat_p, wn, bn, wc, bc, wi, bi)


def _combine(x, agg, cntp, wroot, w4, b, head, n, d, nrel, npass, blk):
    grid = (n // blk,)
    acc_rows = agg.shape[1]
    in_specs = [pl.BlockSpec((blk, d), lambda i: (i, 0))]
    args = [x]
    for r in range(nrel):
        in_specs.append(pl.BlockSpec(
            (npass, blk, SW),
            functools.partial(lambda i, rr: (0, rr * (n // blk) + i, 0), rr=r)))
        args.append(agg)
    for r in range(nrel):
        in_specs.append(pl.BlockSpec(
            (2, blk, 16),
            functools.partial(lambda i, rr: (0, rr * (n // blk) + i, 0), rr=r)))
        args.append(cntp)
    in_specs += [
        pl.BlockSpec((d, d), lambda i: (0, 0)),
        pl.BlockSpec((nrel * d, d), lambda i: (0, 0)),
        pl.BlockSpec((1, d), lambda i: (0, 0)),
    ]
    args += [wroot, w4, b]
    if head is not None:
        wo1, bo1, wo2, bo2 = head
        in_specs += [
            pl.BlockSpec((d, d), lambda i: (0, 0)),
            pl.BlockSpec((1, d), lambda i: (0, 0)),
            pl.BlockSpec((d, d), lambda i: (0, 0)),
            pl.BlockSpec((1, d), lambda i: (0, 0)),
        ]
        args += [wo1, bo1, wo2, bo2]

    body = lambda *refs: _combine_head_body(nrel, npass, head is not None, refs)
    return pl.pallas_call(
        body,
        grid=grid,
        in_specs=in_specs,
        out_specs=pl.BlockSpec((blk, d), lambda i: (i, 0)),
        out_shape=jax.ShapeDtypeStruct((n, d), jnp.float32),
    )(*args)


def _combine_head_body(nrel, npass, with_head, refs):
    (x_ref, *rest) = refs
    a_refs = rest[:nrel]
    c_refs = rest[nrel:2 * nrel]
    if with_head:
        (wroot_ref, w4_ref, b_ref, wo1_ref, bo1_ref, wo2_ref, bo2_ref,
         o_ref) = rest[2 * nrel:]
    else:
        (wroot_ref, w4_ref, b_ref, o_ref) = rest[2 * nrel:]

    acc = jnp.dot(x_ref[...], wroot_ref[...],
                  preferred_element_type=jnp.float32) + b_ref[...]
    w4 = w4_ref[...]
    d = w4.shape[1]
    for r in range(nrel):
        ar = a_refs[r][...]
        m = jnp.dot(ar[0], w4[r * d:r * d + SW],
                    preferred_element_type=jnp.float32)
        for p in range(1, npass):
            m = m + jnp.dot(ar[p], w4[r * d + p * SW:r * d + (p + 1) * SW],
                            preferred_element_type=jnp.float32)
        cp = c_refs[r][...]
        cnt = cp[0, :, 0] + cp[1, :, 0]
        recip = 1.0 / jnp.maximum(cnt, 1.0)
        acc = acc + m * recip[:, None]
    if with_head:
        h = _leaky(jnp.dot(acc, wo1_ref[...],
                           preferred_element_type=jnp.float32) + bo1_ref[...])
        acc = jnp.dot(h, wo2_ref[...],
                      preferred_element_type=jnp.float32) + bo2_ref[...]
    o_ref[...] = acc


def _make_counts_kernel(n, acc_rows, e2):
    t2 = e2 // (NC * NS)
    nb = t2 // EBLK
    rpt = acc_rows // NS
    mesh = plsc.VectorSubcoreMesh(core_axis_name="c", subcore_axis_name="s")

    @functools.partial(
        pl.kernel, mesh=mesh,
        out_type=jax.ShapeDtypeStruct((NC, acc_rows, 16), jnp.float32),
        scratch_types=(
            [pltpu.VMEM((EBLK,), jnp.int32),
             pltpu.VMEM((EBLK,), jnp.int32),
             pltpu.VMEM((CH, 16), jnp.float32)]
            + [pltpu.VMEM((CH,), jnp.int32) for _ in range(NCH)]
            + [pltpu.VMEM_SHARED((acc_rows, 16), jnp.float32)]
        ),
        compiler_params=pltpu.CompilerParams(use_tc_tiling_on_sc=False),
    )
    def counts_kernel(dst_hbm, et_hbm, zeros_hbm, out_hbm, dstv, etv, ones_v,
                      *rest):
        idxs = rest[:NCH]
        acc = rest[NCH]
        c = lax.axis_index("c")
        s = lax.axis_index("s")

        def fill_row(i, _):
            ones_v[i, :] = jnp.ones((16,), jnp.float32)
            return 0
        lax.fori_loop(0, CH, fill_row, 0)

        row0 = s * rpt
        pltpu.sync_copy(zeros_hbm, acc.at[pl.ds(row0, rpt)])
        plsc.subcore_barrier()

        base0 = c * (e2 // 2) + s * t2

        def block(b, _):
            eb = base0 + b * EBLK
            pltpu.sync_copy(dst_hbm.at[pl.ds(eb, EBLK)], dstv)
            pltpu.sync_copy(et_hbm.at[pl.ds(eb, EBLK)], etv)
            for m in range(NCH):
                def cw(k, _):
                    o = m * CH + k * 16
                    dv = dstv[pl.ds(o, 16)]
                    tv = etv[pl.ds(o, 16)]
                    idxs[m][pl.ds(k * 16, 16)] = tv * n + dv
                    return 0
                lax.fori_loop(0, CH // 16, cw, 0)
            for m in range(NCH):
                pltpu.sync_copy(ones_v, acc.at[idxs[m]], add=True)
            return 0
        lax.fori_loop(0, nb, block, 0)
        plsc.subcore_barrier()
        pltpu.sync_copy(acc.at[pl.ds(row0, rpt)],
                        out_hbm.at[c, pl.ds(row0, rpt)])

    return counts_kernel


def _make_segsum_kernel(n, acc_rows, e2, npass):
    tpt = e2 // NS
    nb = tpt // EBLK
    rpt = acc_rows // NS
    mesh = plsc.VectorSubcoreMesh(core_axis_name="c", subcore_axis_name="s")

    @functools.partial(
        pl.kernel, mesh=mesh,
        out_type=jax.ShapeDtypeStruct((npass, acc_rows, SW), jnp.float32),
        scratch_types=(
            [pltpu.VMEM((EBLK,), jnp.int32),
             pltpu.VMEM((EBLK,), jnp.int32),
             pltpu.VMEM((EBLK,), jnp.int32),
             pltpu.VMEM((CH, SW), jnp.float32),
             pltpu.VMEM((CH, SW), jnp.float32)]
            + [pltpu.VMEM((CH,), jnp.int32) for _ in range(NCH)]
            + [pltpu.VMEM((CH,), jnp.int32) for _ in range(NCH)]
            + [pltpu.VMEM_SHARED((acc_rows, SW), jnp.float32),
               pltpu.SemaphoreType.DMA, pltpu.SemaphoreType.DMA]
        ),
        compiler_params=pltpu.CompilerParams(use_tc_tiling_on_sc=False),
    )
    def segsum_kernel(x_hbm, src_hbm, dst_hbm, et_hbm, zeros_hbm, out_hbm,
                      srcv, dstv, etv, rows_a, rows_b, *rest):
        idxg = rest[:NCH]
        idxs = rest[NCH:2 * NCH]
        acc = rest[2 * NCH]
        sems = (rest[2 * NCH + 1], rest[2 * NCH + 2])
        bufs = (rows_a, rows_b)
        c = lax.axis_index("c")
        s = lax.axis_index("s")

        row0 = s * rpt
        base0 = s * tpt

        for q in range(npass // NC):
            p = c + NC * q
            pltpu.sync_copy(zeros_hbm, acc.at[pl.ds(row0, rpt)])
            plsc.subcore_barrier()

            def block(b, _):
                eb = base0 + b * EBLK
                pltpu.sync_copy(src_hbm.at[pl.ds(eb, EBLK)], srcv)
                pltpu.sync_copy(dst_hbm.at[pl.ds(eb, EBLK)], dstv)
                pltpu.sync_copy(et_hbm.at[pl.ds(eb, EBLK)], etv)
                for m in range(NCH):
                    def cw(k, _):
                        o = m * CH + k * 16
                        sv = srcv[pl.ds(o, 16)]
                        dv = dstv[pl.ds(o, 16)]
                        tv = etv[pl.ds(o, 16)]
                        idxg[m][pl.ds(k * 16, 16)] = sv * npass + p
                        idxs[m][pl.ds(k * 16, 16)] = tv * n + dv
                        return 0
                    lax.fori_loop(0, CH // 16, cw, 0)
                handles = [None] * NCH
                handles[0] = pltpu.async_copy(
                    x_hbm.at[idxg[0]], bufs[0], sems[0])
                for m in range(NCH):
                    handles[m].wait()
                    if m + 1 < NCH:
                        handles[m + 1] = pltpu.async_copy(
                            x_hbm.at[idxg[m + 1]],
                            bufs[(m + 1) % 2], sems[(m + 1) % 2])
                    pltpu.sync_copy(bufs[m % 2], acc.at[idxs[m]], add=True)
                return 0
            lax.fori_loop(0, nb, block, 0)
            plsc.subcore_barrier()
            pltpu.sync_copy(acc.at[pl.ds(row0, rpt)],
                            out_hbm.at[p, pl.ds(row0, rpt)])
            plsc.subcore_barrier()

    return segsum_kernel


def kernel(des, tweet, num_prop, cat_prop, edge_index, edge_type,
           W_num, b_num, W_cat, b_cat, W_in, b_in,
           W_rel, W_root, b_rgcn, W_out1, b_out1, W_out2, b_out2):
    n = num_prop.shape[0]
    e = edge_index.shape[1]
    nrel = W_rel.shape[0]
    d = W_in.shape[0]
    npass = d // SW
    blk = 1000

    unit = NC * NS * EBLK
    e2 = ((e + unit - 1) // unit) * unit
    rows_per_tile = -(-(nrel * n + 1) // NS)
    rows_per_tile = ((rows_per_tile + ZR - 1) // ZR) * ZR
    acc_rows = rows_per_tile * NS

    src = edge_index[0]
    dst = edge_index[1]
    et = edge_type
    padn = e2 - e
    if padn:
        zpad = jnp.zeros((padn,), jnp.int32)
        src = jnp.concatenate([src, zpad])
        dst = jnp.concatenate([dst, zpad])
        et = jnp.concatenate([et, jnp.full((padn,), nrel, jnp.int32)])

    np8 = jnp.pad(num_prop, ((0, 0), (0, 8 - num_prop.shape[1])))
    cp16 = jnp.pad(cat_prop, ((0, 0), (0, 16 - cat_prop.shape[1])))
    wn8 = jnp.pad(W_num, ((0, 8 - W_num.shape[0]), (0, 0)))
    wc16 = jnp.pad(W_cat, ((0, 16 - W_cat.shape[0]), (0, 0)))

    x0 = _input_mlp(np8, cp16, wn8, b_num[None, :], wc16, b_cat[None, :],
                    W_in, b_in[None, :], n, d, blk)

    zeros_hbm = jnp.zeros((rows_per_tile, SW), jnp.float32)
    counts_kernel = _make_counts_kernel(n, acc_rows, e2)
    cntp = counts_kernel(dst, et, zeros_hbm)

    segsum_kernel = _make_segsum_kernel(n, acc_rows, e2, npass)
    w4 = W_rel.reshape(nrel * d, d)

    agg1 = segsum_kernel(x0.reshape(n * npass, SW), src, dst, et, zeros_hbm)
    x1 = _combine(x0, agg1, cntp, W_root, w4, b_rgcn[None, :], None,
                  n, d, nrel, npass, blk)

    wo2 = jnp.pad(W_out2, ((0, 0), (0, d - W_out2.shape[1])))
    bo2 = jnp.pad(b_out2, ((0, d - b_out2.shape[0],)))[None, :]
    agg2 = segsum_kernel(x1.reshape(n * npass, SW), src, dst, et, zeros_hbm)
    out128 = _combine(x1, agg2, cntp, W_root, w4, b_rgcn[None, :],
                      (W_out1, b_out1[None, :], wo2, bo2),
                      n, d, nrel, npass, blk)
    return out128[:, :W_out2.shape[1]]


# speedup vs baseline: 6.7046x; 1.3503x over previous
import functools

import jax
import jax.numpy as jnp
from jax import lax
from jax.experimental import pallas as pl
from jax.experimental.pallas import tpu as pltpu
from jax.experimental.pallas import tpu_sc as plsc

NC = 2
NS = 16
EBLK = 2048
CH = 128
NCH = EBLK // CH
ZR = 800
SW = 16


def _leaky(x):
    return jnp.where(x >= 0, x, 0.01 * x)


def _mlp_body(np_ref, cp_ref, wn_ref, bn_ref, wc_ref, bc_ref, wi_ref, bi_ref,
              o_ref):
    nf = _leaky(jnp.dot(np_ref[...], wn_ref[...],
                        preferred_element_type=jnp.float32) + bn_ref[...])
    cf = _leaky(jnp.dot(cp_ref[...], wc_ref[...],
                        preferred_element_type=jnp.float32) + bc_ref[...])
    wi = wi_ref[...]
    h = wi.shape[0] // 2
    x = (jnp.dot(nf, wi[:h], preferred_element_type=jnp.float32)
         + jnp.dot(cf, wi[h:], preferred_element_type=jnp.float32)
         + bi_ref[...])
    o_ref[...] = _leaky(x)


def _input_mlp(num_p, cat_p, wn, bn, wc, bc, wi, bi, n, d, blk):
    grid = (n // blk,)
    fn = num_p.shape[1]
    fc = cat_p.shape[1]
    dh = d // 2
    return pl.pallas_call(
        _mlp_body,
        grid=grid,
        in_specs=[
            pl.BlockSpec((blk, fn), lambda i: (i, 0)),
            pl.BlockSpec((blk, fc), lambda i: (i, 0)),
            pl.BlockSpec((fn, dh), lambda i: (0, 0)),
            pl.BlockSpec((1, dh), lambda i: (0, 0)),
            pl.BlockSpec((fc, dh), lambda i: (0, 0)),
            pl.BlockSpec((1, dh), lambda i: (0, 0)),
            pl.BlockSpec((d, d), lambda i: (0, 0)),
            pl.BlockSpec((1, d), lambda i: (0, 0)),
        ],
        out_specs=pl.BlockSpec((blk, d), lambda i: (i, 0)),
        out_shape=jax.ShapeDtypeStruct((n, d), jnp.float32),
    )(num_p, cat_p, wn, bn, wc, bc, wi, bi)


def _combine(x, agg, cntp, wroot, w4, b, head, n, d, nrel, npass, blk):
    grid = (n // blk,)
    acc_rows = agg.shape[1]
    in_specs = [pl.BlockSpec((blk, d), lambda i: (i, 0))]
    args = [x]
    for r in range(nrel):
        in_specs.append(pl.BlockSpec(
            (npass, blk, SW),
            functools.partial(lambda i, rr: (0, rr * (n // blk) + i, 0), rr=r)))
        args.append(agg)
    for r in range(nrel):
        in_specs.append(pl.BlockSpec(
            (2, blk, 16),
            functools.partial(lambda i, rr: (0, rr * (n // blk) + i, 0), rr=r)))
        args.append(cntp)
    in_specs += [
        pl.BlockSpec((d, d), lambda i: (0, 0)),
        pl.BlockSpec((nrel * d, d), lambda i: (0, 0)),
        pl.BlockSpec((1, d), lambda i: (0, 0)),
    ]
    args += [wroot, w4, b]
    if head is not None:
        wo1, bo1, wo2, bo2 = head
        in_specs += [
            pl.BlockSpec((d, d), lambda i: (0, 0)),
            pl.BlockSpec((1, d), lambda i: (0, 0)),
            pl.BlockSpec((d, d), lambda i: (0, 0)),
            pl.BlockSpec((1, d), lambda i: (0, 0)),
        ]
        args += [wo1, bo1, wo2, bo2]

    body = lambda *refs: _combine_head_body(nrel, npass, head is not None, refs)
    return pl.pallas_call(
        body,
        grid=grid,
        in_specs=in_specs,
        out_specs=pl.BlockSpec((blk, d), lambda i: (i, 0)),
        out_shape=jax.ShapeDtypeStruct((n, d), jnp.float32),
    )(*args)


def _combine_head_body(nrel, npass, with_head, refs):
    (x_ref, *rest) = refs
    a_refs = rest[:nrel]
    c_refs = rest[nrel:2 * nrel]
    if with_head:
        (wroot_ref, w4_ref, b_ref, wo1_ref, bo1_ref, wo2_ref, bo2_ref,
         o_ref) = rest[2 * nrel:]
    else:
        (wroot_ref, w4_ref, b_ref, o_ref) = rest[2 * nrel:]

    acc = jnp.dot(x_ref[...], wroot_ref[...],
                  preferred_element_type=jnp.float32) + b_ref[...]
    w4 = w4_ref[...]
    d = w4.shape[1]
    for r in range(nrel):
        ar = a_refs[r][...]
        m = jnp.dot(ar[0], w4[r * d:r * d + SW],
                    preferred_element_type=jnp.float32)
        for p in range(1, npass):
            m = m + jnp.dot(ar[p], w4[r * d + p * SW:r * d + (p + 1) * SW],
                            preferred_element_type=jnp.float32)
        cp = c_refs[r][...]
        cnt = cp[0, :, 0] + cp[1, :, 0]
        recip = 1.0 / jnp.maximum(cnt, 1.0)
        acc = acc + m * recip[:, None]
    if with_head:
        h = _leaky(jnp.dot(acc, wo1_ref[...],
                           preferred_element_type=jnp.float32) + bo1_ref[...])
        acc = jnp.dot(h, wo2_ref[...],
                      preferred_element_type=jnp.float32) + bo2_ref[...]
    o_ref[...] = acc


def _make_counts_kernel(n, acc_rows, e2):
    t2 = e2 // (NC * NS)
    nb = t2 // EBLK
    rpt = acc_rows // NS
    mesh = plsc.VectorSubcoreMesh(core_axis_name="c", subcore_axis_name="s")

    @functools.partial(
        pl.kernel, mesh=mesh,
        out_type=jax.ShapeDtypeStruct((NC, acc_rows, 16), jnp.float32),
        scratch_types=(
            [pltpu.VMEM((EBLK,), jnp.int32),
             pltpu.VMEM((EBLK,), jnp.int32),
             pltpu.VMEM((CH, 16), jnp.float32)]
            + [pltpu.VMEM((CH,), jnp.int32) for _ in range(NCH)]
            + [pltpu.VMEM_SHARED((acc_rows, 16), jnp.float32)]
        ),
        compiler_params=pltpu.CompilerParams(use_tc_tiling_on_sc=False),
    )
    def counts_kernel(dst_hbm, et_hbm, zeros_hbm, out_hbm, dstv, etv, ones_v,
                      *rest):
        idxs = rest[:NCH]
        acc = rest[NCH]
        c = lax.axis_index("c")
        s = lax.axis_index("s")

        def fill_row(i, _):
            ones_v[i, :] = jnp.ones((16,), jnp.float32)
            return 0
        lax.fori_loop(0, CH, fill_row, 0)

        row0 = s * rpt
        pltpu.sync_copy(zeros_hbm, acc.at[pl.ds(row0, rpt)])
        plsc.subcore_barrier()

        base0 = c * (e2 // 2) + s * t2

        def block(b, _):
            eb = base0 + b * EBLK
            pltpu.sync_copy(dst_hbm.at[pl.ds(eb, EBLK)], dstv)
            pltpu.sync_copy(et_hbm.at[pl.ds(eb, EBLK)], etv)
            for m in range(NCH):
                def cw(k, _):
                    o = m * CH + k * 16
                    dv = dstv[pl.ds(o, 16)]
                    tv = etv[pl.ds(o, 16)]
                    idxs[m][pl.ds(k * 16, 16)] = tv * n + dv
                    return 0
                lax.fori_loop(0, CH // 16, cw, 0)
            for m in range(NCH):
                pltpu.sync_copy(ones_v, acc.at[idxs[m]], add=True)
            return 0
        lax.fori_loop(0, nb, block, 0)
        plsc.subcore_barrier()
        pltpu.sync_copy(acc.at[pl.ds(row0, rpt)],
                        out_hbm.at[c, pl.ds(row0, rpt)])

    return counts_kernel


def _make_segsum_kernel(n, acc_rows, e2, npass):
    tpt = e2 // NS
    nb = tpt // EBLK
    rpt = acc_rows // NS
    mesh = plsc.VectorSubcoreMesh(core_axis_name="c", subcore_axis_name="s")

    @functools.partial(
        pl.kernel, mesh=mesh,
        out_type=jax.ShapeDtypeStruct((npass, acc_rows, SW), jnp.float32),
        scratch_types=(
            [pltpu.VMEM((EBLK,), jnp.int32),
             pltpu.VMEM((EBLK,), jnp.int32),
             pltpu.VMEM((EBLK,), jnp.int32)]
            + [pltpu.VMEM((CH, SW), jnp.float32) for _ in range(NCH)]
            + [pltpu.VMEM((CH,), jnp.int32) for _ in range(NCH)]
            + [pltpu.VMEM((CH,), jnp.int32) for _ in range(NCH)]
            + [pltpu.VMEM_SHARED((acc_rows, SW), jnp.float32),
               pltpu.SemaphoreType.DMA, pltpu.SemaphoreType.DMA]
        ),
        compiler_params=pltpu.CompilerParams(use_tc_tiling_on_sc=False),
    )
    def segsum_kernel(x_hbm, src_hbm, dst_hbm, et_hbm, zeros_hbm, out_hbm,
                      srcv, dstv, etv, *rest):
        bufs = rest[:NCH]
        idxg = rest[NCH:2 * NCH]
        idxs = rest[2 * NCH:3 * NCH]
        acc = rest[3 * NCH]
        semg = rest[3 * NCH + 1]
        semsc = rest[3 * NCH + 2]
        c = lax.axis_index("c")
        s = lax.axis_index("s")

        row0 = s * rpt
        base0 = s * tpt

        for q in range(npass // NC):
            p = c + NC * q
            pltpu.sync_copy(zeros_hbm, acc.at[pl.ds(row0, rpt)])
            plsc.subcore_barrier()

            def block(b, _):
                eb = base0 + b * EBLK
                pltpu.sync_copy(src_hbm.at[pl.ds(eb, EBLK)], srcv)
                pltpu.sync_copy(dst_hbm.at[pl.ds(eb, EBLK)], dstv)
                pltpu.sync_copy(et_hbm.at[pl.ds(eb, EBLK)], etv)
                for m in range(NCH):
                    def cw(k, _):
                        o = m * CH + k * 16
                        sv = srcv[pl.ds(o, 16)]
                        dv = dstv[pl.ds(o, 16)]
                        tv = etv[pl.ds(o, 16)]
                        idxg[m][pl.ds(k * 16, 16)] = sv * npass + p
                        idxs[m][pl.ds(k * 16, 16)] = tv * n + dv
                        return 0
                    lax.fori_loop(0, CH // 16, cw, 0)
                gh = [pltpu.async_copy(x_hbm.at[idxg[m]], bufs[m], semg)
                      for m in range(NCH)]
                sh = []
                for m in range(NCH):
                    gh[m].wait()
                    sh.append(pltpu.async_copy(bufs[m], acc.at[idxs[m]],
                                               semsc, add=True))
                for h in sh:
                    h.wait()
                return 0
            lax.fori_loop(0, nb, block, 0)
            plsc.subcore_barrier()
            pltpu.sync_copy(acc.at[pl.ds(row0, rpt)],
                            out_hbm.at[p, pl.ds(row0, rpt)])
            plsc.subcore_barrier()

    return segsum_kernel


def kernel(des, tweet, num_prop, cat_prop, edge_index, edge_type,
           W_num, b_num, W_cat, b_cat, W_in, b_in,
           W_rel, W_root, b_rgcn, W_out1, b_out1, W_out2, b_out2):
    n = num_prop.shape[0]
    e = edge_index.shape[1]
    nrel = W_rel.shape[0]
    d = W_in.shape[0]
    npass = d // SW
    blk = 1000

    unit = NC * NS * EBLK
    e2 = ((e + unit - 1) // unit) * unit
    rows_per_tile = -(-(nrel * n + 1) // NS)
    rows_per_tile = ((rows_per_tile + ZR - 1) // ZR) * ZR
    acc_rows = rows_per_tile * NS

    src = edge_index[0]
    dst = edge_index[1]
    et = edge_type
    padn = e2 - e
    if padn:
        zpad = jnp.zeros((padn,), jnp.int32)
        src = jnp.concatenate([src, zpad])
        dst = jnp.concatenate([dst, zpad])
        et = jnp.concatenate([et, jnp.full((padn,), nrel, jnp.int32)])

    np8 = jnp.pad(num_prop, ((0, 0), (0, 8 - num_prop.shape[1])))
    cp16 = jnp.pad(cat_prop, ((0, 0), (0, 16 - cat_prop.shape[1])))
    wn8 = jnp.pad(W_num, ((0, 8 - W_num.shape[0]), (0, 0)))
    wc16 = jnp.pad(W_cat, ((0, 16 - W_cat.shape[0]), (0, 0)))

    x0 = _input_mlp(np8, cp16, wn8, b_num[None, :], wc16, b_cat[None, :],
                    W_in, b_in[None, :], n, d, blk)

    zeros_hbm = jnp.zeros((rows_per_tile, SW), jnp.float32)
    counts_kernel = _make_counts_kernel(n, acc_rows, e2)
    cntp = counts_kernel(dst, et, zeros_hbm)

    segsum_kernel = _make_segsum_kernel(n, acc_rows, e2, npass)
    w4 = W_rel.reshape(nrel * d, d)

    agg1 = segsum_kernel(x0.reshape(n * npass, SW), src, dst, et, zeros_hbm)
    x1 = _combine(x0, agg1, cntp, W_root, w4, b_rgcn[None, :], None,
                  n, d, nrel, npass, blk)

    wo2 = jnp.pad(W_out2, ((0, 0), (0, d - W_out2.shape[1])))
    bo2 = jnp.pad(b_out2, ((0, d - b_out2.shape[0],)))[None, :]
    agg2 = segsum_kernel(x1.reshape(n * npass, SW), src, dst, et, zeros_hbm)
    out128 = _combine(x1, agg2, cntp, W_root, w4, b_rgcn[None, :],
                      (W_out1, b_out1[None, :], wo2, bo2),
                      n, d, nrel, npass, blk)
    return out128[:, :W_out2.shape[1]]
